# baseline (device time: 62540 ns/iter reference)
import jax
import jax.numpy as jnp
from jax import lax
from jax.experimental import pallas as pl
from jax.experimental.pallas import tpu as pltpu

N_DEV = 4
B_LOC = 2
SQ = 512
SKV = 512
H_LOC = 8
DH = 64
D_MODEL = 768
D_HEADS = H_LOC * DH
D_HALF = D_HEADS // 2
H_HALF = H_LOC // 2


def _body(x_ref, kt_ref, vt_ref, wq_ref, wot_ref, out_ref,
          commL_ref, commR_ref, ctx_ref, bias_ref,
          qsendL, qrecvL, qsendR, qrecvR,
          osendL, orecvL, osendR, orecvR):
    my = lax.axis_index("i")
    left = lax.rem(my + (N_DEV - 1), N_DEV)
    right = lax.rem(my + 1, N_DEV)

    commL_ref[0, 0] = wq_ref[:, :D_HALF]
    commL_ref[0, 1] = wot_ref[:, :D_HALF]
    commR_ref[0, 0] = wq_ref[:, D_HALF:]
    commR_ref[0, 1] = wot_ref[:, D_HALF:]

    barrier_sem = pltpu.get_barrier_semaphore()
    for nbr in (left, right):
        pl.semaphore_signal(barrier_sem, inc=1, device_id=(nbr,),
                            device_id_type=pl.DeviceIdType.MESH)
    pl.semaphore_wait(barrier_sem, 2)

    def fwd(comm_ref, part, send_sems, recv_sems, h, dev):
        return pltpu.make_async_remote_copy(
            src_ref=comm_ref.at[h, part],
            dst_ref=comm_ref.at[h + 1, part],
            send_sem=send_sems.at[h],
            recv_sem=recv_sems.at[h],
            device_id=(dev,),
            device_id_type=pl.DeviceIdType.MESH,
        )

    qrdmas = [None] * (N_DEV - 1)
    ordmas = [None] * (N_DEV - 1)
    for h in range(N_DEV):
        if h >= 1:
            qrdmas[h - 1][0].wait_recv()
            qrdmas[h - 1][1].wait_recv()
        if h < N_DEV - 1:
            ql = fwd(commL_ref, 0, qsendL, qrecvL, h, right)
            qr = fwd(commR_ref, 0, qsendR, qrecvR, h, left)
            ql.start()
            qr.start()
            qrdmas[h] = (ql, qr)
        if h == 0:
            qi = lax.broadcasted_iota(jnp.int32, (SQ, SKV), 0)
            ki = lax.broadcasted_iota(jnp.int32, (SQ, SKV), 1)
            mask = (jnp.abs(qi - ki) <= 128) | (ki < 32) | (qi < 32)
            bias_ref[...] = jnp.where(mask, 0.0, -1e9).astype(jnp.float32)

        originL = lax.rem(my + (N_DEV - h), N_DEV)
        originR = lax.rem(my + h, N_DEV)
        wqL = commL_ref[h, 0]
        wqR = commR_ref[h, 0]
        for b in range(B_LOC):
            qL = lax.dot_general(x_ref[b], wqL, (((1,), (0,)), ((), ())),
                                 preferred_element_type=jnp.float32
                                 ).astype(jnp.bfloat16)
            qR = lax.dot_general(x_ref[b], wqR, (((1,), (0,)), ((), ())),
                                 preferred_element_type=jnp.float32
                                 ).astype(jnp.bfloat16)
            for t in range(H_LOC):
                if t < H_HALF:
                    g = originL * H_LOC + t
                    qt = qL[:, t * DH:(t + 1) * DH]
                else:
                    g = originR * H_LOC + t
                    qt = qR[:, (t - H_HALF) * DH:(t - H_HALF + 1) * DH]
                kt = kt_ref[b, g]
                vt = vt_ref[b, g]
                s = lax.dot_general(qt, kt, (((1,), (1,)), ((), ())),
                                    preferred_element_type=jnp.float32)
                p = jnp.exp(s + bias_ref[...])
                denom = jnp.sum(p, axis=-1, keepdims=True)
                ctx = lax.dot_general(p.astype(jnp.bfloat16), vt,
                                      (((1,), (0,)), ((), ())),
                                      preferred_element_type=jnp.float32)
                ctx = ctx * (1.0 / denom)
                ctx_ref[:, t * DH:(t + 1) * DH] = ctx.astype(jnp.bfloat16)

            if b == 0:
                if h >= 1:
                    ordmas[h - 1][0].wait_recv()
                    ordmas[h - 1][1].wait_recv()
                if h < N_DEV - 1:
                    ol = fwd(commL_ref, 1, osendL, orecvL, h, right)
                    orr = fwd(commR_ref, 1, osendR, orecvR, h, left)
                    ol.start()
                    orr.start()
                    ordmas[h] = (ol, orr)

            wotL = commL_ref[h, 1]
            wotR = commR_ref[h, 1]
            partial = lax.dot_general(ctx_ref[:, :D_HALF], wotL,
                                      (((1,), (1,)), ((), ())),
                                      preferred_element_type=jnp.float32)
            partial = partial + lax.dot_general(
                ctx_ref[:, D_HALF:], wotR, (((1,), (1,)), ((), ())),
                preferred_element_type=jnp.float32)
            if h == 0:
                out_ref[b] = partial
            else:
                out_ref[b] = out_ref[b] + partial

    for pair in qrdmas + ordmas:
        if pair is not None:
            pair[0].wait_send()
            pair[1].wait_send()


def kernel(x, Wq, K_ext, V_ext, Wo):
    my = lax.axis_index("i")
    xb = (x * 0.125).astype(jnp.bfloat16)
    wq = Wq.astype(jnp.bfloat16)
    wot = Wo.T.astype(jnp.bfloat16)
    k_loc = lax.dynamic_slice_in_dim(K_ext, my * B_LOC, B_LOC, axis=0)
    v_loc = lax.dynamic_slice_in_dim(V_ext, my * B_LOC, B_LOC, axis=0)
    kt = jnp.transpose(k_loc, (0, 2, 1, 3)).astype(jnp.bfloat16)
    vt = jnp.transpose(v_loc, (0, 2, 1, 3)).astype(jnp.bfloat16)

    return pl.pallas_call(
        _body,
        out_shape=jax.ShapeDtypeStruct((B_LOC, SQ, D_MODEL), jnp.float32),
        in_specs=[pl.BlockSpec(memory_space=pltpu.VMEM)] * 5,
        out_specs=pl.BlockSpec(memory_space=pltpu.VMEM),
        scratch_shapes=[
            pltpu.VMEM((N_DEV, 2, D_MODEL, D_HALF), jnp.bfloat16),
            pltpu.VMEM((N_DEV, 2, D_MODEL, D_HALF), jnp.bfloat16),
            pltpu.VMEM((SQ, D_HEADS), jnp.bfloat16),
            pltpu.VMEM((SQ, SKV), jnp.float32),
        ] + [pltpu.SemaphoreType.DMA((N_DEV - 1,))] * 8,
        compiler_params=pltpu.CompilerParams(collective_id=0),
    )(xb, kt, vt, wq, wot)


# device time: 49817 ns/iter; 1.2554x vs baseline; 1.2554x over previous
import jax
import jax.numpy as jnp
from jax import lax
from jax.experimental import pallas as pl
from jax.experimental.pallas import tpu as pltpu

N_DEV = 4
B_LOC = 2
SQ = 512
SKV = 512
H_LOC = 8
DH = 64
D_MODEL = 768
D_HEADS = H_LOC * DH
D_HALF = D_HEADS // 2
H_HALF = H_LOC // 2


def _body(x_ref, kt_ref, vt_ref, wq_ref, wot_ref, out_ref,
          commL_ref, commR_ref, ctx_ref, bias_ref,
          qsendL, qrecvL, qsendR, qrecvR,
          osendL, orecvL, osendR, orecvR):
    my = lax.axis_index("i")
    left = lax.rem(my + (N_DEV - 1), N_DEV)
    right = lax.rem(my + 1, N_DEV)

    commL_ref[0, 0] = wq_ref[:, :D_HALF]
    commL_ref[0, 1] = wot_ref[:, :D_HALF]
    commR_ref[0, 0] = wq_ref[:, D_HALF:]
    commR_ref[0, 1] = wot_ref[:, D_HALF:]

    barrier_sem = pltpu.get_barrier_semaphore()
    for nbr in (left, right):
        pl.semaphore_signal(barrier_sem, inc=1, device_id=(nbr,),
                            device_id_type=pl.DeviceIdType.MESH)
    pl.semaphore_wait(barrier_sem, 2)

    def fwd(comm_ref, part, send_sems, recv_sems, h, dev):
        return pltpu.make_async_remote_copy(
            src_ref=comm_ref.at[h, part],
            dst_ref=comm_ref.at[h + 1, part],
            send_sem=send_sems.at[h],
            recv_sem=recv_sems.at[h],
            device_id=(dev,),
            device_id_type=pl.DeviceIdType.MESH,
        )

    qrdmas = [None] * (N_DEV - 1)
    ordmas = [None] * (N_DEV - 1)
    for h in range(N_DEV):
        if h >= 1:
            qrdmas[h - 1][0].wait_recv()
            qrdmas[h - 1][1].wait_recv()
        if h < N_DEV - 1:
            ql = fwd(commL_ref, 0, qsendL, qrecvL, h, right)
            qr = fwd(commR_ref, 0, qsendR, qrecvR, h, left)
            ql.start()
            qr.start()
            qrdmas[h] = (ql, qr)
        if h == 0:
            qi = lax.broadcasted_iota(jnp.int32, (SQ, SKV), 0)
            ki = lax.broadcasted_iota(jnp.int32, (SQ, SKV), 1)
            mask = (jnp.abs(qi - ki) <= 128) | (ki < 32) | (qi < 32)
            bias_ref[...] = jnp.where(mask, 0.0, -1e9).astype(jnp.float32)

        originL = lax.rem(my + (N_DEV - h), N_DEV)
        originR = lax.rem(my + h, N_DEV)
        wqL = commL_ref[h, 0]
        wqR = commR_ref[h, 0]
        for b in range(B_LOC):
            qL = lax.dot_general(x_ref[b], wqL, (((1,), (0,)), ((), ())),
                                 preferred_element_type=jnp.float32
                                 ).astype(jnp.bfloat16)
            qR = lax.dot_general(x_ref[b], wqR, (((1,), (0,)), ((), ())),
                                 preferred_element_type=jnp.float32
                                 ).astype(jnp.bfloat16)
            for t in range(H_LOC):
                if t < H_HALF:
                    g = originL * H_LOC + t
                    qt = qL[:, t * DH:(t + 1) * DH]
                else:
                    g = originR * H_LOC + t
                    qt = qR[:, (t - H_HALF) * DH:(t - H_HALF + 1) * DH]
                kt = kt_ref[b, g]
                vt = vt_ref[b, g]
                s = lax.dot_general(qt, kt, (((1,), (0,)), ((), ())),
                                    preferred_element_type=jnp.float32)
                p = jnp.exp(s + bias_ref[...])
                denom = jnp.sum(p, axis=-1, keepdims=True)
                ctx = lax.dot_general(p.astype(jnp.bfloat16), vt,
                                      (((1,), (1,)), ((), ())),
                                      preferred_element_type=jnp.float32)
                ctx = ctx * (1.0 / denom)
                ctx_ref[:, t * DH:(t + 1) * DH] = ctx.astype(jnp.bfloat16)

            if b == 0:
                if h >= 1:
                    ordmas[h - 1][0].wait_recv()
                    ordmas[h - 1][1].wait_recv()
                if h < N_DEV - 1:
                    ol = fwd(commL_ref, 1, osendL, orecvL, h, right)
                    orr = fwd(commR_ref, 1, osendR, orecvR, h, left)
                    ol.start()
                    orr.start()
                    ordmas[h] = (ol, orr)

            wotL = commL_ref[h, 1]
            wotR = commR_ref[h, 1]
            partial = lax.dot_general(ctx_ref[:, :D_HALF], wotL,
                                      (((1,), (1,)), ((), ())),
                                      preferred_element_type=jnp.float32)
            partial = partial + lax.dot_general(
                ctx_ref[:, D_HALF:], wotR, (((1,), (1,)), ((), ())),
                preferred_element_type=jnp.float32)
            if h == 0:
                out_ref[b] = partial
            else:
                out_ref[b] = out_ref[b] + partial

    for pair in qrdmas + ordmas:
        if pair is not None:
            pair[0].wait_send()
            pair[1].wait_send()


def kernel(x, Wq, K_ext, V_ext, Wo):
    my = lax.axis_index("i")
    xb = (x * 0.125).astype(jnp.bfloat16)
    wq = Wq.astype(jnp.bfloat16)
    wot = Wo.T.astype(jnp.bfloat16)
    k_loc = lax.dynamic_slice_in_dim(K_ext, my * B_LOC, B_LOC, axis=0)
    v_loc = lax.dynamic_slice_in_dim(V_ext, my * B_LOC, B_LOC, axis=0)
    kt = jnp.transpose(k_loc, (0, 2, 3, 1)).astype(jnp.bfloat16)
    vt = jnp.transpose(v_loc, (0, 2, 3, 1)).astype(jnp.bfloat16)

    return pl.pallas_call(
        _body,
        out_shape=jax.ShapeDtypeStruct((B_LOC, SQ, D_MODEL), jnp.float32),
        in_specs=[pl.BlockSpec(memory_space=pltpu.VMEM)] * 5,
        out_specs=pl.BlockSpec(memory_space=pltpu.VMEM),
        scratch_shapes=[
            pltpu.VMEM((N_DEV, 2, D_MODEL, D_HALF), jnp.bfloat16),
            pltpu.VMEM((N_DEV, 2, D_MODEL, D_HALF), jnp.bfloat16),
            pltpu.VMEM((SQ, D_HEADS), jnp.bfloat16),
            pltpu.VMEM((SQ, SKV), jnp.float32),
        ] + [pltpu.SemaphoreType.DMA((N_DEV - 1,))] * 8,
        compiler_params=pltpu.CompilerParams(collective_id=0),
    )(xb, kt, vt, wq, wot)
